# baseline (speedup 1.0000x reference)
import jax
import jax.numpy as jnp
from jax.experimental import pallas as pl
from jax.experimental.pallas import tpu as pltpu

_R, _C = 4096, 1024
_BR = 512


def _body(p_ref, l_ref, out_ref, acc_ref):
    i = pl.program_id(0)

    @pl.when(i == 0)
    def _init():
        acc_ref[0] = 0.0
        acc_ref[1] = 0.0

    l = l_ref[...]
    p = p_ref[...]
    nan = jnp.isnan(l)
    d = jnp.where(nan, 0.0, p - l)
    acc_ref[0] += jnp.sum(d * d)
    acc_ref[1] += jnp.sum(jnp.where(nan, 0.0, 1.0))

    @pl.when(i == pl.num_programs(0) - 1)
    def _fin():
        out_ref[0] = acc_ref[0] / acc_ref[1]


def kernel(preds, labels):
    p = preds.reshape(_R, _C)
    l = labels.reshape(_R, _C)
    out = pl.pallas_call(
        _body,
        grid=(_R // _BR,),
        in_specs=[
            pl.BlockSpec((_BR, _C), lambda i: (i, 0)),
            pl.BlockSpec((_BR, _C), lambda i: (i, 0)),
        ],
        out_specs=pl.BlockSpec(memory_space=pltpu.SMEM),
        out_shape=jax.ShapeDtypeStruct((1,), jnp.float32),
        scratch_shapes=[pltpu.SMEM((2,), jnp.float32)],
    )(p, l)
    return out[0]


# speedup vs baseline: 1.0561x; 1.0561x over previous
import jax
import jax.numpy as jnp
from jax.experimental import pallas as pl
from jax.experimental.pallas import tpu as pltpu

_N, _L, _C = 16, 4096, 64
_BL = 1024


def _body(p_ref, l_ref, out_ref, acc_ref):
    i = pl.program_id(0)
    j = pl.program_id(1)
    step = i * pl.num_programs(1) + j

    @pl.when(step == 0)
    def _init():
        acc_ref[0] = 0.0
        acc_ref[1] = 0.0

    l = l_ref[...]
    p = p_ref[...]
    nan = jnp.isnan(l)
    d = jnp.where(nan, 0.0, p - l)
    acc_ref[0] += jnp.sum(d * d)
    acc_ref[1] += jnp.sum(jnp.where(nan, 0.0, 1.0))

    @pl.when(step == pl.num_programs(0) * pl.num_programs(1) - 1)
    def _fin():
        out_ref[0] = acc_ref[0] / acc_ref[1]


def kernel(preds, labels):
    out = pl.pallas_call(
        _body,
        grid=(_N, _L // _BL),
        in_specs=[
            pl.BlockSpec((1, _BL, _C), lambda i, j: (i, j, 0)),
            pl.BlockSpec((1, _BL, _C), lambda i, j: (i, j, 0)),
        ],
        out_specs=pl.BlockSpec(memory_space=pltpu.SMEM),
        out_shape=jax.ShapeDtypeStruct((1,), jnp.float32),
        scratch_shapes=[pltpu.SMEM((2,), jnp.float32)],
    )(preds, labels)
    return out[0]
